# baseline (speedup 1.0000x reference)
import jax
import jax.numpy as jnp
from jax.experimental import pallas as pl
from jax.experimental.pallas import tpu as pltpu

_TOTAL_ROWS = 2 * 4096
_SRC_OFFSET = 1 * 4096
_B = 256


def _copy_body(x_ref, out_ref):
    out_ref[...] = x_ref[...]


def kernel(x):
    x2 = x.reshape(4 * 4096, 4096)
    out = pl.pallas_call(
        _copy_body,
        grid=(_TOTAL_ROWS // _B,),
        in_specs=[
            pl.BlockSpec((_B, 4096), lambda i: (i + _SRC_OFFSET // _B, 0))
        ],
        out_specs=pl.BlockSpec((_B, 4096), lambda i: (i, 0)),
        out_shape=jax.ShapeDtypeStruct((_TOTAL_ROWS, 4096), jnp.float32),
    )(x2)
    return out.reshape(2, 4096, 4096)


# speedup vs baseline: 1.0199x; 1.0199x over previous
import jax
import jax.numpy as jnp
from jax.experimental import pallas as pl
from jax.experimental.pallas import tpu as pltpu

_TOTAL_ROWS = 2 * 4096
_SRC_OFFSET = 1 * 4096
_B = 512


def _copy_body(x_ref, out_ref):
    out_ref[...] = x_ref[...]


def kernel(x):
    x2 = x.reshape(4 * 4096, 4096)
    out = pl.pallas_call(
        _copy_body,
        grid=(_TOTAL_ROWS // _B,),
        in_specs=[
            pl.BlockSpec((_B, 4096), lambda i: (i + _SRC_OFFSET // _B, 0))
        ],
        out_specs=pl.BlockSpec((_B, 4096), lambda i: (i, 0)),
        out_shape=jax.ShapeDtypeStruct((_TOTAL_ROWS, 4096), jnp.float32),
        compiler_params=pltpu.CompilerParams(
            dimension_semantics=("parallel",)
        ),
    )(x2)
    return out.reshape(2, 4096, 4096)


# speedup vs baseline: 1.0330x; 1.0129x over previous
import jax
import jax.numpy as jnp
from jax.experimental import pallas as pl
from jax.experimental.pallas import tpu as pltpu

_TOTAL_ROWS = 2 * 4096
_SRC_OFFSET = 1 * 4096
_BUF_ROWS = 512
_R = 6
_D = 2

_CHUNKS = [64, 192, 256] + [512] * 14 + [256, 192, 64]
_OFFS = [0]
for _c in _CHUNKS:
    _OFFS.append(_OFFS[-1] + _c)
assert _OFFS[-1] == _TOTAL_ROWS
_N = len(_CHUNKS)


def _copy_body(x_hbm, out_hbm, *scratch):
    bufs = scratch[:_R]
    in_sems = scratch[_R]
    out_sems = scratch[_R + 1]

    def mk_in(i):
        return pltpu.make_async_copy(
            x_hbm.at[pl.ds(_SRC_OFFSET + _OFFS[i], _CHUNKS[i])],
            bufs[i % _R].at[pl.ds(0, _CHUNKS[i])],
            in_sems.at[i % _R],
        )

    def mk_out(i):
        return pltpu.make_async_copy(
            bufs[i % _R].at[pl.ds(0, _CHUNKS[i])],
            out_hbm.at[pl.ds(_OFFS[i], _CHUNKS[i])],
            out_sems.at[i % _R],
        )

    for i in range(_R):
        mk_in(i).start()

    waited_out = set()
    for i in range(_N):
        mk_in(i).wait()
        mk_out(i).start()
        j = i - _D
        if j >= 0 and j + _R < _N:
            mk_out(j).wait()
            waited_out.add(j)
            mk_in(j + _R).start()
    for i in range(_N):
        if i not in waited_out:
            mk_out(i).wait()


def kernel(x):
    x2 = x.reshape(4 * 4096, 4096)
    out = pl.pallas_call(
        _copy_body,
        in_specs=[pl.BlockSpec(memory_space=pl.ANY)],
        out_specs=pl.BlockSpec(memory_space=pl.ANY),
        out_shape=jax.ShapeDtypeStruct((_TOTAL_ROWS, 4096), jnp.float32),
        scratch_shapes=(
            [pltpu.VMEM((_BUF_ROWS, 4096), jnp.float32) for _ in range(_R)]
            + [pltpu.SemaphoreType.DMA((_R,)), pltpu.SemaphoreType.DMA((_R,))]
        ),
    )(x2)
    return out.reshape(2, 4096, 4096)
